# baseline (device time: 27123 ns/iter reference)
import jax
import jax.numpy as jnp
from jax import lax
from jax.experimental import pallas as pl
from jax.experimental.pallas import tpu as pltpu

N_DEV = 4
B, SQ, SKV = 2, 128, 128
H_PER = 4
DH = 64
D_MODEL = 512
HD = H_PER * DH


def kernel(x, Wq, K_ext, V_ext, Wo):
    K2 = K_ext.reshape(B, SKV, HD)
    V2 = V_ext.reshape(B, SKV, HD)

    def body(x_ref, wq_ref, k_ref, v_ref, wo_ref, out_ref,
             comm_ref, send_sems, recv_sems):
        my = lax.axis_index("i")
        left = lax.rem(my + N_DEV - 1, N_DEV)
        right = lax.rem(my + 1, N_DEV)

        barrier_sem = pltpu.get_barrier_semaphore()
        for nbr in (left, right):
            pl.semaphore_signal(
                barrier_sem, inc=1,
                device_id=(nbr,), device_id_type=pl.DeviceIdType.MESH,
            )
        pl.semaphore_wait(barrier_sem, 2)

        wq_slice = wq_ref[:, pl.ds(my * HD, HD)].astype(jnp.bfloat16)
        wo_slice = wo_ref[pl.ds(my * HD, HD), :].astype(jnp.bfloat16)

        row_blk = lax.broadcasted_iota(jnp.int32, (SQ, SKV), 0) // 64
        col_blk = lax.broadcasted_iota(jnp.int32, (SQ, SKV), 1) // 64
        mask = col_blk <= row_blk

        for b in range(B):
            xb = x_ref[b].astype(jnp.bfloat16)
            qb = jnp.dot(xb, wq_slice,
                         preferred_element_type=jnp.float32)
            kb = k_ref[b].astype(jnp.bfloat16)
            vb = v_ref[b].astype(jnp.bfloat16)

            ctxs = []
            for h in range(H_PER):
                q = qb[:, h * DH:(h + 1) * DH].astype(jnp.bfloat16)
                k = kb[:, h * DH:(h + 1) * DH]
                v = vb[:, h * DH:(h + 1) * DH]
                scores = jnp.dot(q, k.T,
                                 preferred_element_type=jnp.float32) * 0.125
                scores = jnp.where(mask, scores, -1e9)
                m = jnp.max(scores, axis=-1, keepdims=True)
                w = jnp.exp(scores - m)
                w = w / jnp.sum(w, axis=-1, keepdims=True)
                ctxs.append(jnp.dot(w.astype(jnp.bfloat16), v,
                                    preferred_element_type=jnp.float32))
            ctx = jnp.concatenate(ctxs, axis=1)
            partial = jnp.dot(ctx.astype(jnp.bfloat16), wo_slice,
                              preferred_element_type=jnp.float32)
            out_ref[b] = partial
            comm_ref[0, b] = partial.astype(jnp.bfloat16)

        for h in range(N_DEV - 1):
            rdma = pltpu.make_async_remote_copy(
                src_ref=comm_ref.at[h],
                dst_ref=comm_ref.at[h + 1],
                send_sem=send_sems.at[h],
                recv_sem=recv_sems.at[h],
                device_id=(right,),
                device_id_type=pl.DeviceIdType.MESH,
            )
            rdma.start()
            rdma.wait()
            out_ref[:] = out_ref[:] + comm_ref[h + 1].astype(jnp.float32)

    return pl.pallas_call(
        body,
        out_shape=jax.ShapeDtypeStruct((B, SQ, D_MODEL), jnp.float32),
        in_specs=[pl.BlockSpec(memory_space=pltpu.VMEM)] * 5,
        out_specs=pl.BlockSpec(memory_space=pltpu.VMEM),
        scratch_shapes=[
            pltpu.VMEM((N_DEV, B, SQ, D_MODEL), jnp.bfloat16),
            pltpu.SemaphoreType.DMA((N_DEV - 1,)),
            pltpu.SemaphoreType.DMA((N_DEV - 1,)),
        ],
        compiler_params=pltpu.CompilerParams(collective_id=0),
    )(x, Wq, K2, V2, Wo)


# device time: 17950 ns/iter; 1.5110x vs baseline; 1.5110x over previous
import jax
import jax.numpy as jnp
from jax import lax
from jax.experimental import pallas as pl
from jax.experimental.pallas import tpu as pltpu

N_DEV = 4
B, SQ, SKV = 2, 128, 128
H_PER = 4
DH = 64
D_MODEL = 512
HD = H_PER * DH


def kernel(x, Wq, K_ext, V_ext, Wo):
    K2 = K_ext.reshape(B, SKV, HD)
    V2 = V_ext.reshape(B, SKV, HD)

    def body(x_ref, wq_ref, k_ref, v_ref, wo_ref, out_ref,
             comm_ref, acc_ref, send_sems, recv_sems):
        my = lax.axis_index("i")
        right = lax.rem(my + 1, N_DEV)
        opp = lax.rem(my + 2, N_DEV)
        left = lax.rem(my + 3, N_DEV)

        barrier_sem = pltpu.get_barrier_semaphore()
        for nbr in (left, right, opp):
            pl.semaphore_signal(
                barrier_sem, inc=1,
                device_id=(nbr,), device_id_type=pl.DeviceIdType.MESH,
            )
        pl.semaphore_wait(barrier_sem, N_DEV - 1)

        wq_slice = wq_ref[:, pl.ds(my * HD, HD)].astype(jnp.bfloat16)

        row_blk = lax.broadcasted_iota(jnp.int32, (SQ, SKV), 0) // 64
        col_blk = lax.broadcasted_iota(jnp.int32, (SQ, SKV), 1) // 64
        mask = col_blk <= row_blk

        xf = x_ref[...].reshape(B * SQ, D_MODEL).astype(jnp.bfloat16)
        qf = jnp.dot(xf, wq_slice,
                     preferred_element_type=jnp.float32)

        dn_qkT = (((1,), (1,)), ((), ()))
        for b in range(B):
            qb = qf[b * SQ:(b + 1) * SQ]
            kb = k_ref[b].astype(jnp.bfloat16)
            vb = v_ref[b].astype(jnp.bfloat16)
            ctxs = []
            for h in range(H_PER):
                q = qb[:, h * DH:(h + 1) * DH].astype(jnp.bfloat16)
                k = kb[:, h * DH:(h + 1) * DH]
                v = vb[:, h * DH:(h + 1) * DH]
                scores = lax.dot_general(
                    q, k, dn_qkT,
                    preferred_element_type=jnp.float32) * 0.125
                scores = jnp.where(mask, scores, -1e9)
                m = jnp.max(scores, axis=-1, keepdims=True)
                w = jnp.exp(scores - m)
                w = w / jnp.sum(w, axis=-1, keepdims=True)
                ctxs.append(jnp.dot(w.astype(jnp.bfloat16), v,
                                    preferred_element_type=jnp.float32))
            comm_ref[0, b] = jnp.concatenate(ctxs, axis=1).astype(jnp.bfloat16)

        def mk(target, dst_slot, i):
            return pltpu.make_async_remote_copy(
                src_ref=comm_ref.at[0],
                dst_ref=comm_ref.at[dst_slot],
                send_sem=send_sems.at[i],
                recv_sem=recv_sems.at[i],
                device_id=(target,),
                device_id_type=pl.DeviceIdType.MESH,
            )

        r_right = mk(right, 3, 0)
        r_left = mk(left, 1, 1)
        r_opp = mk(opp, 2, 2)
        r_right.start()
        r_left.start()
        r_opp.start()

        wo_my = wo_ref[pl.ds(my * HD, HD), :].astype(jnp.bfloat16)
        ctx_me = comm_ref[0].reshape(B * SQ, HD)
        acc_ref[...] = jnp.dot(ctx_me, wo_my,
                               preferred_element_type=jnp.float32)

        for rdma, o in ((r_left, 1), (r_right, 3), (r_opp, 2)):
            rdma.wait_recv()
            src_dev = lax.rem(my + o, N_DEV)
            wo_o = wo_ref[pl.ds(src_dev * HD, HD), :].astype(jnp.bfloat16)
            ctx_o = comm_ref[o].reshape(B * SQ, HD)
            acc_ref[...] = acc_ref[...] + jnp.dot(
                ctx_o, wo_o, preferred_element_type=jnp.float32)

        out_ref[...] = acc_ref[...].reshape(B, SQ, D_MODEL)

        r_right.wait_send()
        r_left.wait_send()
        r_opp.wait_send()

    return pl.pallas_call(
        body,
        out_shape=jax.ShapeDtypeStruct((B, SQ, D_MODEL), jnp.float32),
        in_specs=[pl.BlockSpec(memory_space=pltpu.VMEM)] * 5,
        out_specs=pl.BlockSpec(memory_space=pltpu.VMEM),
        scratch_shapes=[
            pltpu.VMEM((N_DEV, B, SQ, HD), jnp.bfloat16),
            pltpu.VMEM((B * SQ, D_MODEL), jnp.float32),
            pltpu.SemaphoreType.DMA((3,)),
            pltpu.SemaphoreType.DMA((3,)),
        ],
        compiler_params=pltpu.CompilerParams(collective_id=0),
    )(x, Wq, K2, V2, Wo)


# device time: 16066 ns/iter; 1.6882x vs baseline; 1.1173x over previous
import jax
import jax.numpy as jnp
from jax import lax
from jax.experimental import pallas as pl
from jax.experimental.pallas import tpu as pltpu

N_DEV = 4
B, SQ, SKV = 2, 128, 128
H_PER = 4
DH = 64
D_MODEL = 512
HD = H_PER * DH


def kernel(x, Wq, K_ext, V_ext, Wo):
    K2 = K_ext.reshape(B, SKV, HD)
    V2 = V_ext.reshape(B, SKV, HD)

    def body(x_ref, wq_ref, k_ref, v_ref, wo_ref, out_ref,
             comm_ref, acc_ref, send_sems, recv_sems):
        my = lax.axis_index("i")
        right = lax.rem(my + 1, N_DEV)
        opp = lax.rem(my + 2, N_DEV)
        left = lax.rem(my + 3, N_DEV)

        barrier_sem = pltpu.get_barrier_semaphore()
        for nbr in (left, right, opp):
            pl.semaphore_signal(
                barrier_sem, inc=1,
                device_id=(nbr,), device_id_type=pl.DeviceIdType.MESH,
            )

        wq_slice = wq_ref[:, pl.ds(my * HD, HD)].astype(jnp.bfloat16)

        row_blk = lax.broadcasted_iota(jnp.int32, (SQ, SKV), 0) // 64
        col_blk = lax.broadcasted_iota(jnp.int32, (SQ, SKV), 1) // 64
        mask = (col_blk <= row_blk)[None]

        xf = x_ref[...].reshape(B * SQ, D_MODEL).astype(jnp.bfloat16)
        qf = jnp.dot(xf, wq_slice,
                     preferred_element_type=jnp.float32)

        pl.semaphore_wait(barrier_sem, N_DEV - 1)

        def mk(b, target, dst_slot, i):
            return pltpu.make_async_remote_copy(
                src_ref=comm_ref.at[0, b],
                dst_ref=comm_ref.at[dst_slot, b],
                send_sem=send_sems.at[b, i],
                recv_sem=recv_sems.at[b, i],
                device_id=(target,),
                device_id_type=pl.DeviceIdType.MESH,
            )

        rdmas = [[mk(b, right, 3, 0), mk(b, left, 1, 1), mk(b, opp, 2, 2)]
                 for b in range(B)]

        dn_qkT = (((2,), (2,)), ((0,), (0,)))
        dn_wv = (((2,), (1,)), ((0,), (0,)))
        for b in range(B):
            qb = qf[b * SQ:(b + 1) * SQ]
            kb = k_ref[b].astype(jnp.bfloat16)
            vb = v_ref[b].astype(jnp.bfloat16)
            qs = jnp.stack([qb[:, h * DH:(h + 1) * DH] for h in range(H_PER)]
                           ).astype(jnp.bfloat16)
            ks = jnp.stack([kb[:, h * DH:(h + 1) * DH] for h in range(H_PER)])
            vs = jnp.stack([vb[:, h * DH:(h + 1) * DH] for h in range(H_PER)])
            s = lax.dot_general(qs, ks, dn_qkT,
                                preferred_element_type=jnp.float32) * 0.125
            w = jnp.exp(jnp.where(mask, s, -1e9))
            w = w / jnp.sum(w, axis=-1, keepdims=True)
            ctx = lax.dot_general(w.astype(jnp.bfloat16), vs, dn_wv,
                                  preferred_element_type=jnp.float32)
            for h in range(H_PER):
                comm_ref[0, b, :, h * DH:(h + 1) * DH] = (
                    ctx[h].astype(jnp.bfloat16))
            for r in rdmas[b]:
                r.start()

        wo_my = wo_ref[pl.ds(my * HD, HD), :].astype(jnp.bfloat16)
        ctx_me = comm_ref[0].reshape(B * SQ, HD)
        acc_ref[...] = jnp.dot(ctx_me, wo_my,
                               preferred_element_type=jnp.float32)

        for i, o in ((1, 1), (0, 3), (2, 2)):
            for b in range(B):
                rdmas[b][i].wait_recv()
            src_dev = lax.rem(my + o, N_DEV)
            wo_o = wo_ref[pl.ds(src_dev * HD, HD), :].astype(jnp.bfloat16)
            ctx_o = comm_ref[o].reshape(B * SQ, HD)
            acc_ref[...] = acc_ref[...] + jnp.dot(
                ctx_o, wo_o, preferred_element_type=jnp.float32)

        out_ref[...] = acc_ref[...].reshape(B, SQ, D_MODEL)

        for b in range(B):
            for r in rdmas[b]:
                r.wait_send()

    return pl.pallas_call(
        body,
        out_shape=jax.ShapeDtypeStruct((B, SQ, D_MODEL), jnp.float32),
        in_specs=[pl.BlockSpec(memory_space=pltpu.VMEM)] * 5,
        out_specs=pl.BlockSpec(memory_space=pltpu.VMEM),
        scratch_shapes=[
            pltpu.VMEM((N_DEV, B, SQ, HD), jnp.bfloat16),
            pltpu.VMEM((B * SQ, D_MODEL), jnp.float32),
            pltpu.SemaphoreType.DMA((B, 3)),
            pltpu.SemaphoreType.DMA((B, 3)),
        ],
        compiler_params=pltpu.CompilerParams(collective_id=0),
    )(x, Wq, K2, V2, Wo)
